# baseline (device time: 132552 ns/iter reference)
import jax
import jax.numpy as jnp
from jax import lax
from jax.experimental import pallas as pl
from jax.experimental.pallas import tpu as pltpu

N_DEV = 8
NH = N_DEV - 1
B, SQ, D = 4, 256, 1024
T = B * SQ
C = T // N_DEV
HEADS, DH = 8, 128
SCALE = 0.08838834764831843


def _body(x_ref, wq_ref, wo_ref, wk_ref, wv_ref, out_ref,
          partial_ref, sbuf, rbuf,
          rs_send, rs_recv, ag_send, ag_recv):
    my = lax.axis_index("i")
    left = lax.rem(my + N_DEV - 1, N_DEV)
    right = lax.rem(my + 1, N_DEV)

    barrier = pltpu.get_barrier_semaphore()
    for nbr in (left, right):
        pl.semaphore_signal(barrier, inc=1, device_id=(nbr,),
                            device_id_type=pl.DeviceIdType.MESH)
    pl.semaphore_wait(barrier, 2)

    xv = x_ref[:, :]
    q = jnp.dot(xv, wq_ref[:, :], preferred_element_type=jnp.float32)
    k = jnp.dot(xv, wk_ref[:, :], preferred_element_type=jnp.float32)
    v = jnp.dot(xv, wv_ref[:, :], preferred_element_type=jnp.float32)

    rows = []
    for b in range(B):
        qb = q[b * SQ:(b + 1) * SQ, :]
        kb = k[b * SQ:(b + 1) * SQ, :]
        vb = v[b * SQ:(b + 1) * SQ, :]
        heads = []
        for h in range(HEADS):
            qh = qb[:, h * DH:(h + 1) * DH]
            kh = kb[:, h * DH:(h + 1) * DH]
            vh = vb[:, h * DH:(h + 1) * DH]
            s = jnp.dot(qh, kh.T, preferred_element_type=jnp.float32) * SCALE
            m = jnp.max(s, axis=-1, keepdims=True)
            p = jnp.exp(s - m)
            l = jnp.sum(p, axis=-1, keepdims=True)
            heads.append(jnp.dot(p, vh, preferred_element_type=jnp.float32) / l)
        rows.append(jnp.concatenate(heads, axis=1))
    attn = jnp.concatenate(rows, axis=0)
    partial_ref[:, :] = jnp.dot(attn, wo_ref[:, :],
                                preferred_element_type=jnp.float32)

    for h in range(NH):
        c_send = lax.rem(my + N_DEV - h, N_DEV)
        if h == 0:
            sbuf[0] = partial_ref[pl.ds(c_send * C, C), :]
        rdma = pltpu.make_async_remote_copy(
            src_ref=sbuf.at[h],
            dst_ref=rbuf.at[h],
            send_sem=rs_send.at[h],
            recv_sem=rs_recv.at[h],
            device_id=(right,),
            device_id_type=pl.DeviceIdType.MESH,
        )
        rdma.start()
        rdma.wait()
        c_recv = lax.rem(my + N_DEV - h - 1, N_DEV)
        acc = rbuf[h] + partial_ref[pl.ds(c_recv * C, C), :]
        if h < NH - 1:
            sbuf[h + 1] = acc
        else:
            out_ref[pl.ds(c_recv * C, C), :] = acc

    for g in range(NH):
        c_send = lax.rem(my + 1 + N_DEV - g, N_DEV)
        c_recv = lax.rem(my + N_DEV - g, N_DEV)
        send = pltpu.make_async_remote_copy(
            src_ref=out_ref.at[pl.ds(c_send * C, C), :],
            dst_ref=out_ref.at[pl.ds(c_send * C, C), :],
            send_sem=ag_send.at[g],
            recv_sem=ag_recv.at[g],
            device_id=(right,),
            device_id_type=pl.DeviceIdType.MESH,
        )
        send.start()
        recv = pltpu.make_async_remote_copy(
            src_ref=out_ref.at[pl.ds(c_recv * C, C), :],
            dst_ref=out_ref.at[pl.ds(c_recv * C, C), :],
            send_sem=ag_send.at[g],
            recv_sem=ag_recv.at[g],
            device_id=(left,),
            device_id_type=pl.DeviceIdType.MESH,
        )
        send.wait_send()
        recv.wait_recv()


def kernel(x, Wq, Wo, Wk, Wv):
    x2 = x.reshape(T, D)
    out = pl.pallas_call(
        _body,
        out_shape=jax.ShapeDtypeStruct((T, D), jnp.float32),
        in_specs=[pl.BlockSpec(memory_space=pltpu.VMEM)] * 5,
        out_specs=pl.BlockSpec(memory_space=pltpu.VMEM),
        scratch_shapes=[
            pltpu.VMEM((T, D), jnp.float32),
            pltpu.VMEM((NH, C, D), jnp.float32),
            pltpu.VMEM((NH, C, D), jnp.float32),
            pltpu.SemaphoreType.DMA((NH,)),
            pltpu.SemaphoreType.DMA((NH,)),
            pltpu.SemaphoreType.DMA((NH,)),
            pltpu.SemaphoreType.DMA((NH,)),
        ],
        compiler_params=pltpu.CompilerParams(collective_id=0),
    )(x2, Wq, Wo, Wk, Wv)
    return out.reshape(B, SQ, D)


# device time: 69393 ns/iter; 1.9102x vs baseline; 1.9102x over previous
import jax
import jax.numpy as jnp
from jax import lax
from jax.experimental import pallas as pl
from jax.experimental.pallas import tpu as pltpu

N_DEV = 8
B, SQ, D = 4, 256, 1024
T = B * SQ
HEADS, DH = 8, 128
SCALE = 0.08838834764831843

MASKS = (1, 3, 4)
PART_DIMS = ((0, 1, 2), (1, 2, 0), (2, 0, 1))
R0 = (0, 384, 768)
RR = (384, 384, 256)


def _body(x_ref, wq_ref, wo_ref, wk_ref, wv_ref, out_ref,
          partial_ref,
          rb00, rb01, rb02, rb10, rb11, rb12, rb20, rb21, rb22,
          rs_send, rs_recv, ag_send, ag_recv):
    rbuf = ((rb00, rb01, rb02), (rb10, rb11, rb12), (rb20, rb21, rb22))

    my = lax.axis_index("i")
    r = jnp.bitwise_and(my, 3)
    yb = jnp.right_shift(r, 1)
    xb = jnp.bitwise_xor(yb, jnp.bitwise_and(r, 1))
    zb = jnp.right_shift(my, 2)
    sides = (xb, yb, zb)
    partners = tuple(jnp.bitwise_xor(my, m) for m in MASKS)

    barrier = pltpu.get_barrier_semaphore()
    for nbr in partners:
        pl.semaphore_signal(barrier, inc=1, device_id=(nbr,),
                            device_id_type=pl.DeviceIdType.MESH)
    pl.semaphore_wait(barrier, 3)

    xv = x_ref[:, :]
    q = jnp.dot(xv, wq_ref[:, :], preferred_element_type=jnp.float32)
    k = jnp.dot(xv, wk_ref[:, :], preferred_element_type=jnp.float32)
    v = jnp.dot(xv, wv_ref[:, :], preferred_element_type=jnp.float32)

    rows = []
    for b in range(B):
        qb = q[b * SQ:(b + 1) * SQ, :]
        kb = k[b * SQ:(b + 1) * SQ, :]
        vb = v[b * SQ:(b + 1) * SQ, :]
        heads = []
        for h in range(HEADS):
            qh = qb[:, h * DH:(h + 1) * DH]
            kh = kb[:, h * DH:(h + 1) * DH]
            vh = vb[:, h * DH:(h + 1) * DH]
            s = jnp.dot(qh, kh.T, preferred_element_type=jnp.float32) * SCALE
            m = jnp.max(s, axis=-1, keepdims=True)
            p = jnp.exp(s - m)
            l = jnp.sum(p, axis=-1, keepdims=True)
            heads.append(jnp.dot(p, vh, preferred_element_type=jnp.float32) / l)
        rows.append(jnp.concatenate(heads, axis=1))
    attn = jnp.concatenate(rows, axis=0)
    partial_ref[:, :] = jnp.dot(attn, wo_ref[:, :],
                                preferred_element_type=jnp.float32)

    off = [jnp.int32(R0[p]) for p in range(3)]
    ln = list(RR)
    for s in range(3):
        started = []
        for p in range(3):
            d = PART_DIMS[p][s]
            bbit = sides[d]
            half = ln[p] // 2
            send_off = off[p] + (1 - bbit) * half
            rdma = pltpu.make_async_remote_copy(
                src_ref=partial_ref.at[pl.ds(send_off, half), :],
                dst_ref=rbuf[p][s],
                send_sem=rs_send.at[p, s],
                recv_sem=rs_recv.at[p, s],
                device_id=(partners[d],),
                device_id_type=pl.DeviceIdType.MESH,
            )
            rdma.start()
            started.append((rdma, bbit, half))
        for p in range(3):
            rdma, bbit, half = started[p]
            rdma.wait()
            keep_off = off[p] + bbit * half
            partial_ref[pl.ds(keep_off, half), :] = (
                partial_ref[pl.ds(keep_off, half), :] + rbuf[p][s][:, :])
            off[p] = keep_off
            ln[p] = half

    for p in range(3):
        out_ref[pl.ds(off[p], ln[p]), :] = partial_ref[pl.ds(off[p], ln[p]), :]

    for s in range(3):
        started = []
        for p in range(3):
            d = PART_DIMS[p][2 - s]
            bbit = sides[d]
            L = ln[p]
            rdma = pltpu.make_async_remote_copy(
                src_ref=out_ref.at[pl.ds(off[p], L), :],
                dst_ref=out_ref.at[pl.ds(off[p], L), :],
                send_sem=ag_send.at[p, s],
                recv_sem=ag_recv.at[p, s],
                device_id=(partners[d],),
                device_id_type=pl.DeviceIdType.MESH,
            )
            rdma.start()
            started.append((rdma, bbit, L))
        for p in range(3):
            rdma, bbit, L = started[p]
            rdma.wait()
            off[p] = off[p] - bbit * L
            ln[p] = 2 * L


def kernel(x, Wq, Wo, Wk, Wv):
    x2 = x.reshape(T, D)
    rs_bufs = [
        pltpu.VMEM((RR[p] >> (s + 1), D), jnp.float32)
        for p in range(3) for s in range(3)
    ]
    out = pl.pallas_call(
        _body,
        out_shape=jax.ShapeDtypeStruct((T, D), jnp.float32),
        in_specs=[pl.BlockSpec(memory_space=pltpu.VMEM)] * 5,
        out_specs=pl.BlockSpec(memory_space=pltpu.VMEM),
        scratch_shapes=[
            pltpu.VMEM((T, D), jnp.float32),
            *rs_bufs,
            pltpu.SemaphoreType.DMA((3, 3)),
            pltpu.SemaphoreType.DMA((3, 3)),
            pltpu.SemaphoreType.DMA((3, 3)),
            pltpu.SemaphoreType.DMA((3, 3)),
        ],
        compiler_params=pltpu.CompilerParams(collective_id=0),
    )(x2, Wq, Wo, Wk, Wv)
    return out.reshape(B, SQ, D)


# device time: 63928 ns/iter; 2.0735x vs baseline; 1.0855x over previous
import jax
import jax.numpy as jnp
from jax import lax
from jax.experimental import pallas as pl
from jax.experimental.pallas import tpu as pltpu

N_DEV = 8
B, SQ, D = 4, 256, 1024
T = B * SQ
HEADS, DH = 8, 128
SCALE = 0.08838834764831843

MASKS = (1, 3, 4)
PART_DIMS = ((0, 1, 2), (1, 2, 0), (2, 0, 1))
R0 = (0, 384, 768)
RR = (384, 384, 256)


def _body(x_ref, wq_ref, wo_ref, wk_ref, wv_ref, out_ref,
          partial_ref,
          rb00, rb01, rb02, rb10, rb11, rb12, rb20, rb21, rb22,
          rs_send, rs_recv, ag_send, ag_recv):
    rbuf = ((rb00, rb01, rb02), (rb10, rb11, rb12), (rb20, rb21, rb22))

    my = lax.axis_index("i")
    r = jnp.bitwise_and(my, 3)
    yb = jnp.right_shift(r, 1)
    xb = jnp.bitwise_xor(yb, jnp.bitwise_and(r, 1))
    zb = jnp.right_shift(my, 2)
    sides = (xb, yb, zb)
    partners = tuple(jnp.bitwise_xor(my, m) for m in MASKS)

    barrier = pltpu.get_barrier_semaphore()
    for nbr in partners:
        pl.semaphore_signal(barrier, inc=1, device_id=(nbr,),
                            device_id_type=pl.DeviceIdType.MESH)
    pl.semaphore_wait(barrier, 3)

    def attn_batch(b):
        xb_ = x_ref[b * SQ:(b + 1) * SQ, :]
        qb = jnp.dot(xb_, wq_ref[:, :], preferred_element_type=jnp.float32)
        kb = jnp.dot(xb_, wk_ref[:, :], preferred_element_type=jnp.float32)
        vb = jnp.dot(xb_, wv_ref[:, :], preferred_element_type=jnp.float32)
        heads = []
        for h in range(HEADS):
            qh = qb[:, h * DH:(h + 1) * DH]
            kh = kb[:, h * DH:(h + 1) * DH]
            vh = vb[:, h * DH:(h + 1) * DH]
            s = jnp.dot(qh, kh.T, preferred_element_type=jnp.float32) * SCALE
            m = jnp.max(s, axis=-1, keepdims=True)
            p = jnp.exp(s - m)
            l = jnp.sum(p, axis=-1, keepdims=True)
            heads.append(jnp.dot(p, vh, preferred_element_type=jnp.float32) / l)
        return jnp.concatenate(heads, axis=1)

    off = [jnp.int32(R0[p]) for p in range(3)]
    ln = list(RR)
    st = [None, None, None]

    def make_rs(p, s):
        d = PART_DIMS[p][s]
        bbit = sides[d]
        half = ln[p] // 2
        send_off = off[p] + (1 - bbit) * half
        rdma = pltpu.make_async_remote_copy(
            src_ref=partial_ref.at[pl.ds(send_off, half), :],
            dst_ref=rbuf[p][s],
            send_sem=rs_send.at[p, s],
            recv_sem=rs_recv.at[p, s],
            device_id=(partners[d],),
            device_id_type=pl.DeviceIdType.MESH,
        )
        rdma.start()
        return rdma, bbit, half

    def make_ag(p, s):
        d = PART_DIMS[p][2 - s]
        bbit = sides[d]
        L = ln[p]
        rdma = pltpu.make_async_remote_copy(
            src_ref=out_ref.at[pl.ds(off[p], L), :],
            dst_ref=out_ref.at[pl.ds(off[p], L), :],
            send_sem=ag_send.at[p, s],
            recv_sem=ag_recv.at[p, s],
            device_id=(partners[d],),
            device_id_type=pl.DeviceIdType.MESH,
        )
        rdma.start()
        return rdma, bbit, L

    a0 = attn_batch(0)
    a1 = attn_batch(1)
    partial_ref[pl.ds(0, 384), :] = jnp.dot(
        jnp.concatenate([a0, a1[:128]], axis=0), wo_ref[:, :],
        preferred_element_type=jnp.float32)
    st[0] = make_rs(0, 0)
    a2 = attn_batch(2)
    partial_ref[pl.ds(384, 384), :] = jnp.dot(
        jnp.concatenate([a1[128:], a2], axis=0), wo_ref[:, :],
        preferred_element_type=jnp.float32)
    st[1] = make_rs(1, 0)
    a3 = attn_batch(3)
    partial_ref[pl.ds(768, 256), :] = jnp.dot(
        a3, wo_ref[:, :], preferred_element_type=jnp.float32)
    st[2] = make_rs(2, 0)

    for step in range(6):
        for p in range(3):
            rdma, bbit, sz = st[p]
            rdma.wait()
            if step < 3:
                keep_off = off[p] + bbit * sz
                partial_ref[pl.ds(keep_off, sz), :] = (
                    partial_ref[pl.ds(keep_off, sz), :]
                    + rbuf[p][step][:, :])
                off[p] = keep_off
                ln[p] = sz
                if step == 2:
                    out_ref[pl.ds(off[p], ln[p]), :] = (
                        partial_ref[pl.ds(off[p], ln[p]), :])
            else:
                off[p] = off[p] - bbit * sz
                ln[p] = 2 * sz
            if step < 2:
                st[p] = make_rs(p, step + 1)
            elif step < 5:
                st[p] = make_ag(p, step - 2)


def kernel(x, Wq, Wo, Wk, Wv):
    x2 = x.reshape(T, D)
    rs_bufs = [
        pltpu.VMEM((RR[p] >> (s + 1), D), jnp.float32)
        for p in range(3) for s in range(3)
    ]
    out = pl.pallas_call(
        _body,
        out_shape=jax.ShapeDtypeStruct((T, D), jnp.float32),
        in_specs=[pl.BlockSpec(memory_space=pltpu.VMEM)] * 5,
        out_specs=pl.BlockSpec(memory_space=pltpu.VMEM),
        scratch_shapes=[
            pltpu.VMEM((T, D), jnp.float32),
            *rs_bufs,
            pltpu.SemaphoreType.DMA((3, 3)),
            pltpu.SemaphoreType.DMA((3, 3)),
            pltpu.SemaphoreType.DMA((3, 3)),
            pltpu.SemaphoreType.DMA((3, 3)),
        ],
        compiler_params=pltpu.CompilerParams(collective_id=0),
    )(x2, Wq, Wo, Wk, Wv)
    return out.reshape(B, SQ, D)


# device time: 62674 ns/iter; 2.1149x vs baseline; 1.0200x over previous
import jax
import jax.numpy as jnp
from jax import lax
from jax.experimental import pallas as pl
from jax.experimental.pallas import tpu as pltpu

N_DEV = 8
B, SQ, D = 4, 256, 1024
T = B * SQ
C = T // N_DEV
HEADS, DH = 8, 128
SCALE = 0.08838834764831843


def _body(x_ref, wq_ref, wo_ref, wk_ref, wv_ref, out_ref,
          pbuf, rbuf, gb, rs_send, rs_recv, ag_send, ag_recv):
    my = lax.axis_index("i")

    barrier = pltpu.get_barrier_semaphore()
    for j in range(1, N_DEV):
        pl.semaphore_signal(
            barrier, inc=1,
            device_id=(lax.rem(my + j, N_DEV),),
            device_id_type=pl.DeviceIdType.MESH)
    pl.semaphore_wait(barrier, N_DEV - 1)

    def attn_batch(b):
        xb_ = x_ref[b * SQ:(b + 1) * SQ, :]
        qb = jnp.dot(xb_, wq_ref[:, :], preferred_element_type=jnp.float32)
        kb = jnp.dot(xb_, wk_ref[:, :], preferred_element_type=jnp.float32)
        vb = jnp.dot(xb_, wv_ref[:, :], preferred_element_type=jnp.float32)
        heads = []
        for h in range(HEADS):
            qh = qb[:, h * DH:(h + 1) * DH]
            kh = kb[:, h * DH:(h + 1) * DH]
            vh = vb[:, h * DH:(h + 1) * DH]
            s = jnp.dot(qh, kh.T, preferred_element_type=jnp.float32) * SCALE
            m = jnp.max(s, axis=-1, keepdims=True)
            p = jnp.exp(s - m)
            l = jnp.sum(p, axis=-1, keepdims=True)
            heads.append(jnp.dot(p, vh, preferred_element_type=jnp.float32) / l)
        return jnp.concatenate(heads, axis=1)

    rs_rdmas = []
    for b in range(B):
        ab = attn_batch(b)
        pbuf[pl.ds(b * SQ, SQ), :] = jnp.dot(
            ab, wo_ref[:, :], preferred_element_type=jnp.float32
        ).astype(jnp.bfloat16)
        for c in (2 * b, 2 * b + 1):
            jj = lax.rem(my - c + N_DEV, N_DEV)
            rdma = pltpu.make_async_remote_copy(
                src_ref=pbuf.at[pl.ds(c * C, C), :],
                dst_ref=rbuf.at[jj],
                send_sem=rs_send.at[c],
                recv_sem=rs_recv.at[jj],
                device_id=(c,),
                device_id_type=pl.DeviceIdType.MESH,
            )
            @pl.when(jj != 0)
            def _(rdma=rdma):
                rdma.start()
            rs_rdmas.append((c, jj, rdma))

    for j in range(1, N_DEV):
        pltpu.make_async_remote_copy(
            src_ref=rbuf.at[j], dst_ref=rbuf.at[j],
            send_sem=rs_send.at[0], recv_sem=rs_recv.at[j],
            device_id=(my,), device_id_type=pl.DeviceIdType.MESH,
        ).wait_recv()
    red = pbuf[pl.ds(my * C, C), :].astype(jnp.float32)
    for j in range(1, N_DEV):
        red = red + rbuf[j].astype(jnp.float32)
    gb[pl.ds(my * C, C), :] = red.astype(jnp.bfloat16)

    ag_rdmas = []
    for j in range(1, N_DEV):
        rdma = pltpu.make_async_remote_copy(
            src_ref=gb.at[pl.ds(my * C, C), :],
            dst_ref=gb.at[pl.ds(my * C, C), :],
            send_sem=ag_send.at[j],
            recv_sem=ag_recv.at[j],
            device_id=(lax.rem(my - j + N_DEV, N_DEV),),
            device_id_type=pl.DeviceIdType.MESH,
        )
        rdma.start()
        ag_rdmas.append(rdma)

    for c, jj, rdma in rs_rdmas:
        @pl.when(jj != 0)
        def _(rdma=rdma):
            rdma.wait_send()

    for j in range(1, N_DEV):
        pltpu.make_async_remote_copy(
            src_ref=gb.at[pl.ds(0, C), :], dst_ref=gb.at[pl.ds(0, C), :],
            send_sem=ag_send.at[0], recv_sem=ag_recv.at[j],
            device_id=(my,), device_id_type=pl.DeviceIdType.MESH,
        ).wait_recv()
    for rdma in ag_rdmas:
        rdma.wait_send()

    out_ref[:, :] = gb[:, :].astype(jnp.float32)


def kernel(x, Wq, Wo, Wk, Wv):
    x2 = x.reshape(T, D)
    out = pl.pallas_call(
        _body,
        out_shape=jax.ShapeDtypeStruct((T, D), jnp.float32),
        in_specs=[pl.BlockSpec(memory_space=pltpu.VMEM)] * 5,
        out_specs=pl.BlockSpec(memory_space=pltpu.VMEM),
        scratch_shapes=[
            pltpu.VMEM((T, D), jnp.bfloat16),
            pltpu.VMEM((N_DEV, C, D), jnp.bfloat16),
            pltpu.VMEM((T, D), jnp.bfloat16),
            pltpu.SemaphoreType.DMA((N_DEV,)),
            pltpu.SemaphoreType.DMA((N_DEV,)),
            pltpu.SemaphoreType.DMA((N_DEV,)),
            pltpu.SemaphoreType.DMA((N_DEV,)),
        ],
        compiler_params=pltpu.CompilerParams(collective_id=0),
    )(x2, Wq, Wo, Wk, Wv)
    return out.reshape(B, SQ, D)


# device time: 49063 ns/iter; 2.7017x vs baseline; 1.2774x over previous
import jax
import jax.numpy as jnp
from jax import lax
from jax.experimental import pallas as pl
from jax.experimental.pallas import tpu as pltpu

N_DEV = 8
B, SQ, D = 4, 256, 1024
T = B * SQ
SC = SQ // N_DEV
HEADS, DH = 8, 128
SCALE = 0.08838834764831843


def _body(x_ref, wq_ref, wo_ref, wk_ref, wv_ref, out_ref,
          pbuf, rbuf, gbuf, agbuf, rs_send, rs_recv, ag_send, ag_recv):
    my = lax.axis_index("i")

    barrier = pltpu.get_barrier_semaphore()
    for j in range(1, N_DEV):
        pl.semaphore_signal(
            barrier, inc=1,
            device_id=(lax.rem(my + j, N_DEV),),
            device_id_type=pl.DeviceIdType.MESH)
    pl.semaphore_wait(barrier, N_DEV - 1)

    def attn_batch(b):
        xb_ = x_ref[b * SQ:(b + 1) * SQ, :]
        qb = jnp.dot(xb_, wq_ref[:, :], preferred_element_type=jnp.float32)
        kb = jnp.dot(xb_, wk_ref[:, :], preferred_element_type=jnp.float32)
        vb = jnp.dot(xb_, wv_ref[:, :], preferred_element_type=jnp.float32)
        heads = []
        for h in range(HEADS):
            qh = qb[:, h * DH:(h + 1) * DH]
            kh = kb[:, h * DH:(h + 1) * DH]
            vh = vb[:, h * DH:(h + 1) * DH]
            s = jnp.dot(qh, kh.T, preferred_element_type=jnp.float32) * SCALE
            m = jnp.max(s, axis=-1, keepdims=True)
            p = jnp.exp(s - m)
            l = jnp.sum(p, axis=-1, keepdims=True)
            heads.append(jnp.dot(p, vh, preferred_element_type=jnp.float32) / l)
        return jnp.concatenate(heads, axis=1)

    rs_rdmas = []
    ag_rdmas = []

    def rs_send_batch(b):
        for d in range(N_DEV):
            jj = lax.rem(my - d + N_DEV, N_DEV)
            rdma = pltpu.make_async_remote_copy(
                src_ref=pbuf.at[pl.ds(b * SQ + d * SC, SC), :],
                dst_ref=rbuf.at[b, jj],
                send_sem=rs_send.at[b, d],
                recv_sem=rs_recv.at[b, jj],
                device_id=(d,),
                device_id_type=pl.DeviceIdType.MESH,
            )
            @pl.when(jj != 0)
            def _(rdma=rdma):
                rdma.start()
            rs_rdmas.append((jj, rdma))

    def reduce_and_broadcast(b):
        for j in range(1, N_DEV):
            pltpu.make_async_remote_copy(
                src_ref=rbuf.at[b, j], dst_ref=rbuf.at[b, j],
                send_sem=rs_send.at[b, 0], recv_sem=rs_recv.at[b, j],
                device_id=(my,), device_id_type=pl.DeviceIdType.MESH,
            ).wait_recv()
        own = pl.ds(b * SQ + my * SC, SC)
        red = pbuf[own, :].astype(jnp.float32)
        for j in range(1, N_DEV):
            red = red + rbuf[b, j].astype(jnp.float32)
        out_ref[own, :] = red
        gbuf[b, :, :] = red.astype(jnp.bfloat16)
        for j in range(1, N_DEV):
            rdma = pltpu.make_async_remote_copy(
                src_ref=gbuf.at[b],
                dst_ref=agbuf.at[b, j],
                send_sem=ag_send.at[b, j],
                recv_sem=ag_recv.at[b, j],
                device_id=(lax.rem(my - j + N_DEV, N_DEV),),
                device_id_type=pl.DeviceIdType.MESH,
            )
            rdma.start()
            ag_rdmas.append(rdma)

    for b in range(B):
        ab = attn_batch(b)
        pbuf[pl.ds(b * SQ, SQ), :] = jnp.dot(
            ab, wo_ref[:, :], preferred_element_type=jnp.float32
        ).astype(jnp.bfloat16)
        rs_send_batch(b)
        if b >= 1:
            reduce_and_broadcast(b - 1)
    reduce_and_broadcast(B - 1)

    for b in range(B):
        for j in range(1, N_DEV):
            pltpu.make_async_remote_copy(
                src_ref=agbuf.at[b, j], dst_ref=agbuf.at[b, j],
                send_sem=ag_send.at[b, 0], recv_sem=ag_recv.at[b, j],
                device_id=(my,), device_id_type=pl.DeviceIdType.MESH,
            ).wait_recv()
            d = lax.rem(my + j, N_DEV)
            out_ref[pl.ds(b * SQ + d * SC, SC), :] = (
                agbuf[b, j].astype(jnp.float32))

    for jj, rdma in rs_rdmas:
        @pl.when(jj != 0)
        def _(rdma=rdma):
            rdma.wait_send()
    for rdma in ag_rdmas:
        rdma.wait_send()


def kernel(x, Wq, Wo, Wk, Wv):
    x2 = x.reshape(T, D)
    out = pl.pallas_call(
        _body,
        out_shape=jax.ShapeDtypeStruct((T, D), jnp.float32),
        in_specs=[pl.BlockSpec(memory_space=pltpu.VMEM)] * 5,
        out_specs=pl.BlockSpec(memory_space=pltpu.VMEM),
        scratch_shapes=[
            pltpu.VMEM((T, D), jnp.bfloat16),
            pltpu.VMEM((B, N_DEV, SC, D), jnp.bfloat16),
            pltpu.VMEM((B, SC, D), jnp.bfloat16),
            pltpu.VMEM((B, N_DEV, SC, D), jnp.bfloat16),
            pltpu.SemaphoreType.DMA((B, N_DEV)),
            pltpu.SemaphoreType.DMA((B, N_DEV)),
            pltpu.SemaphoreType.DMA((B, N_DEV)),
            pltpu.SemaphoreType.DMA((B, N_DEV)),
        ],
        compiler_params=pltpu.CompilerParams(collective_id=0),
    )(x2, Wq, Wo, Wk, Wv)
    return out.reshape(B, SQ, D)
